# baseline (speedup 1.0000x reference)
import functools

import numpy as np
import jax
import jax.numpy as jnp
from jax import lax
from jax.experimental import pallas as pl
from jax.experimental.pallas import tpu as pltpu
from jax.experimental.pallas import tpu_sc as plsc

N = 32768
C = 81
D = 3

_NC = 2
_NS = 16
_L = 16
_NW = _NC * _NS
_BPW = N // _NW
_EPW = _BPW * D
_NROW = _EPW // 128

_p = np.arange(N * D)
_chunk, _off = _p // _EPW, _p % _EPW
_SIDX = (_chunk * _BPW + _off % _BPW) * D + _off // _BPW
_SIDX = _SIDX.astype(np.int32).reshape(N * D // 128, 128)

_mesh = plsc.VectorSubcoreMesh(core_axis_name="c", subcore_axis_name="s")


@functools.partial(
    pl.kernel,
    mesh=_mesh,
    out_type=jax.ShapeDtypeStruct((N * D,), jnp.float32),
    scratch_types=[
        pltpu.VMEM((_BPW,), jnp.int32),
        pltpu.VMEM((_NROW, 128), jnp.int32),
        pltpu.VMEM((_NROW, 128), jnp.int32),
        pltpu.VMEM((_EPW,), jnp.float32),
        pltpu.SemaphoreType.DMA,
        pltpu.SemaphoreType.DMA,
    ],
)
def _gather_elems(table_hbm, labels_hbm, sidx_hbm, out_hbm,
                  lab_v, gidx_v, sidx_v, vals_v, gsem, ssem):
    wid = lax.axis_index("s") * _NC + lax.axis_index("c")
    base = wid * _BPW
    pltpu.sync_copy(labels_hbm.at[pl.ds(base, _BPW)], lab_v)
    pltpu.sync_copy(sidx_hbm.at[pl.ds(wid * _NROW, _NROW)], sidx_v)

    iota = lax.iota(jnp.int32, _L)
    for d in range(D):
        for j in range(_BPW // _L):
            lab16 = lab_v[pl.ds(j * _L, _L)]
            row16 = (base + j * _L) + iota
            p = d * _BPW + j * _L
            gidx_v[p // 128, pl.ds(p % 128, _L)] = row16 * (C * D) + lab16 * D + d

    gathers = [
        pltpu.async_copy(
            table_hbm.at[gidx_v.at[j]],
            vals_v.at[pl.ds(j * 128, 128)],
            gsem,
        )
        for j in range(_NROW)
    ]
    scatters = []
    for j in range(_NROW):
        gathers[j].wait()
        scatters.append(
            pltpu.async_copy(
                vals_v.at[pl.ds(j * 128, 128)],
                out_hbm.at[sidx_v.at[j]],
                ssem,
            )
        )
    for s in scatters:
        s.wait()


def kernel(depth_pred, labels):
    table = depth_pred.reshape(N * C * D)
    lab = labels.astype(jnp.int32)
    sidx = jnp.asarray(_SIDX)
    return _gather_elems(table, lab, sidx).reshape(N, D)


# speedup vs baseline: 519.0228x; 519.0228x over previous
import jax
import jax.numpy as jnp
from jax import lax
from jax.experimental import pallas as pl
from jax.experimental.pallas import tpu as pltpu

N = 32768
C = 81
D = 3

_BI = 2048
_NBLK = N // _BI


def _select_kernel(lab_ref, tab_ref, out_ref):
    lab = lab_ref[...]
    cio = lax.broadcasted_iota(jnp.int32, (C, _BI), 0)
    mask = cio == lab
    for d in range(D):
        picked = jnp.where(mask, tab_ref[d], 0.0)
        out_ref[pl.ds(d, 1), :] = jnp.sum(picked, axis=0, keepdims=True)


def kernel(depth_pred, labels):
    table = depth_pred.transpose(2, 1, 0)
    lab2d = labels.astype(jnp.int32).reshape(1, N)
    out_t = pl.pallas_call(
        _select_kernel,
        grid=(_NBLK,),
        in_specs=[
            pl.BlockSpec((1, _BI), lambda b: (0, b)),
            pl.BlockSpec((D, C, _BI), lambda b: (0, 0, b)),
        ],
        out_specs=pl.BlockSpec((D, _BI), lambda b: (0, b)),
        out_shape=jax.ShapeDtypeStruct((D, N), jnp.float32),
        compiler_params=pltpu.CompilerParams(
            dimension_semantics=("parallel",),
        ),
    )(lab2d, table)
    return out_t.T


# speedup vs baseline: 657.8870x; 1.2675x over previous
import jax
import jax.numpy as jnp
from jax import lax
from jax.experimental import pallas as pl
from jax.experimental.pallas import tpu as pltpu

N = 32768
C = 81
D = 3

_BI = 4096
_NBLK = N // _BI


def _select_kernel(lab_ref, tab_ref, out_ref):
    lab = lab_ref[...]
    cio = lax.broadcasted_iota(jnp.int32, (C, _BI), 0)
    mask = cio == lab
    for d in range(D):
        picked = jnp.where(mask, tab_ref[d], 0.0)
        out_ref[pl.ds(d, 1), :] = jnp.sum(picked, axis=0, keepdims=True)


def kernel(depth_pred, labels):
    table = depth_pred.transpose(2, 1, 0)
    lab2d = labels.astype(jnp.int32).reshape(1, N)
    out_t = pl.pallas_call(
        _select_kernel,
        grid=(_NBLK,),
        in_specs=[
            pl.BlockSpec((1, _BI), lambda b: (0, b)),
            pl.BlockSpec((D, C, _BI), lambda b: (0, 0, b)),
        ],
        out_specs=pl.BlockSpec((D, _BI), lambda b: (0, b)),
        out_shape=jax.ShapeDtypeStruct((D, N), jnp.float32),
        compiler_params=pltpu.CompilerParams(
            dimension_semantics=("parallel",),
        ),
    )(lab2d, table)
    return out_t.T


# speedup vs baseline: 701.4182x; 1.0662x over previous
import jax
import jax.numpy as jnp
from jax import lax
from jax.experimental import pallas as pl
from jax.experimental.pallas import tpu as pltpu

N = 32768
C = 81
D = 3

_BI = 8192
_NBLK = N // _BI


def _select_kernel(lab_ref, tab_ref, out_ref):
    lab = lab_ref[...]
    cio = lax.broadcasted_iota(jnp.int32, (C, _BI), 0)
    mask = cio == lab
    for d in range(D):
        picked = jnp.where(mask, tab_ref[d], 0.0)
        out_ref[pl.ds(d, 1), :] = jnp.sum(picked, axis=0, keepdims=True)


def kernel(depth_pred, labels):
    table = depth_pred.transpose(2, 1, 0)
    lab2d = labels.astype(jnp.int32).reshape(1, N)
    out_t = pl.pallas_call(
        _select_kernel,
        grid=(_NBLK,),
        in_specs=[
            pl.BlockSpec((1, _BI), lambda b: (0, b)),
            pl.BlockSpec((D, C, _BI), lambda b: (0, 0, b)),
        ],
        out_specs=pl.BlockSpec((D, _BI), lambda b: (0, b)),
        out_shape=jax.ShapeDtypeStruct((D, N), jnp.float32),
        compiler_params=pltpu.CompilerParams(
            dimension_semantics=("parallel",),
        ),
    )(lab2d, table)
    return out_t.T
